# Initial kernel scaffold; baseline (speedup 1.0000x reference)
import jax
import jax.numpy as jnp
from jax.experimental import pallas as pl


def kernel(xyuv, xyuv_grid):
    raise NotImplementedError("write your pallas kernel here")


# speedup vs baseline: 1.1758x; 1.1758x over previous
import functools

import jax
import jax.numpy as jnp
from jax import lax
from jax.experimental import pallas as pl
from jax.experimental.pallas import tpu as pltpu
from jax.experimental.pallas import tpu_sc as plsc

R, X, Y, U, V = 8, 16, 16, 128, 128
N = 1048576
L = 16
P = 128
G = P // L

_SCALE = (float(X - 1), float(Y - 1), float(U - 1), float(V - 1))
_STRIDE = (Y * U * V, U * V, V, 1)


def _sc_interp(table, xyuv):
    info = plsc.get_sparse_core_info()
    nw = info.num_cores * info.num_subcores
    pts_per_worker = N // nw
    nchunks = pts_per_worker // P
    mesh = plsc.VectorSubcoreMesh(core_axis_name="c", subcore_axis_name="s")

    @functools.partial(
        pl.kernel,
        out_type=jax.ShapeDtypeStruct((N, R), jnp.float32),
        mesh=mesh,
        scratch_types=[
            pltpu.VMEM((P, 4), jnp.float32),
            pltpu.VMEM((16, P), jnp.int32),
            pltpu.VMEM((16, P), jnp.float32),
            pltpu.VMEM((16 * P, R), jnp.float32),
            pltpu.VMEM((P, R), jnp.float32),
            pltpu.SemaphoreType.DMA,
        ],
        compiler_params=pltpu.CompilerParams(
            needs_layout_passes=False, use_tc_tiling_on_sc=False
        ),
    )
    def k(table_h, xyuv_h, out_h, coords, idxbuf, wbuf, rowsbuf, outbuf, gsem):
        wid = lax.axis_index("s") * info.num_cores + lax.axis_index("c")
        lane = lax.broadcasted_iota(jnp.int32, (L,), 0)

        def chunk(i, carry):
            p0 = wid * pts_per_worker + i * P
            pltpu.sync_copy(xyuv_h.at[pl.ds(p0, P)], coords)

            def group_a(g, c2):
                pts = g * L + lane
                fr = []
                i0 = []
                for d in range(4):
                    col = jnp.full((L,), d, jnp.int32)
                    c = plsc.load_gather(coords, [pts, col])
                    norm = c * 2.0 - 1.0
                    pos = (norm + 1.0) * 0.5 * _SCALE[d]
                    idx = pos.astype(jnp.int32)
                    i0.append(idx)
                    fr.append(pos - idx.astype(jnp.float32))
                base = (
                    i0[0] * _STRIDE[0] + i0[1] * _STRIDE[1]
                    + i0[2] * _STRIDE[2] + i0[3]
                )
                w0 = [1.0 - f for f in fr]
                for cx in range(2):
                    wx = fr[0] if cx else w0[0]
                    for cy in range(2):
                        wxy = wx * (fr[1] if cy else w0[1])
                        for cu in range(2):
                            wxyu = wxy * (fr[2] if cu else w0[2])
                            for cv in range(2):
                                corner = cx | (cy << 1) | (cu << 2) | (cv << 3)
                                off = (cx * _STRIDE[0] + cy * _STRIDE[1]
                                       + cu * _STRIDE[2] + cv)
                                idxbuf[corner, pl.ds(g * L, L)] = base + off
                                w = wxyu * (fr[3] if cv else w0[3])
                                wbuf[corner, pl.ds(g * L, L)] = w
                return c2

            lax.fori_loop(0, G, group_a, 0)

            handles = [
                pltpu.async_copy(
                    table_h.at[idxbuf.at[corner]],
                    rowsbuf.at[pl.ds(corner * P, P)],
                    gsem,
                )
                for corner in range(16)
            ]
            for h in handles:
                h.wait()

            def group_b(g, c2):
                pts = g * L + lane
                acc = [jnp.zeros((L,), jnp.float32) for _ in range(R)]
                for corner in range(16):
                    w = wbuf[corner, pl.ds(g * L, L)]
                    row = corner * P + g * L + lane
                    for r in range(R):
                        col = jnp.full((L,), r, jnp.int32)
                        val = plsc.load_gather(rowsbuf, [row, col])
                        acc[r] = acc[r] + w * val
                for r in range(R):
                    col = jnp.full((L,), r, jnp.int32)
                    plsc.store_scatter(outbuf, [pts, col], acc[r])
                return c2

            lax.fori_loop(0, G, group_b, 0)
            pltpu.sync_copy(outbuf, out_h.at[pl.ds(p0, P)])
            return carry

        lax.fori_loop(0, nchunks, chunk, 0)

    return k(table, xyuv)


def kernel(xyuv, xyuv_grid):
    table = jnp.transpose(xyuv_grid[0], (1, 2, 3, 4, 0)).reshape(X * Y * U * V, R)
    return _sc_interp(table, xyuv)


# speedup vs baseline: 1.2626x; 1.0738x over previous
import functools

import jax
import jax.numpy as jnp
from jax import lax
from jax.experimental import pallas as pl
from jax.experimental.pallas import tpu as pltpu
from jax.experimental.pallas import tpu_sc as plsc

R, X, Y, U, V = 8, 16, 16, 128, 128
N = 1048576
L = 16
P = 128
G = P // L

_SCALE = (float(X - 1), float(Y - 1), float(U - 1), float(V - 1))
_STRIDE = (Y * U * V, U * V, V, 1)


def _sc_interp(table, xyuv):
    info = plsc.get_sparse_core_info()
    nw = info.num_cores * info.num_subcores
    pts_per_worker = N // nw
    nchunks = pts_per_worker // P
    mesh = plsc.VectorSubcoreMesh(core_axis_name="c", subcore_axis_name="s")

    @functools.partial(
        pl.kernel,
        out_type=jax.ShapeDtypeStruct((N, R), jnp.float32),
        mesh=mesh,
        scratch_types=[
            pltpu.VMEM((P, 4), jnp.float32),
            pltpu.VMEM((16, P), jnp.int32),
            pltpu.VMEM((16, P), jnp.float32),
            pltpu.VMEM((16 * P, R), jnp.float32),
            pltpu.VMEM((P, R), jnp.float32),
            pltpu.SemaphoreType.DMA,
        ],
        compiler_params=pltpu.CompilerParams(
            needs_layout_passes=False, use_tc_tiling_on_sc=False
        ),
    )
    def k(table_h, xyuv_h, out_h, coords, idxbuf, wbuf, rowsbuf, outbuf, gsem):
        wid = lax.axis_index("s") * info.num_cores + lax.axis_index("c")
        lane = lax.broadcasted_iota(jnp.int32, (L,), 0)

        def chunk(i, carry):
            p0 = wid * pts_per_worker + i * P
            pltpu.sync_copy(xyuv_h.at[pl.ds(p0, P)], coords)

            def group_a(g, c2):
                pts = g * L + lane
                fr = []
                i0 = []
                for d in range(4):
                    col = jnp.full((L,), d, jnp.int32)
                    c = plsc.load_gather(coords, [pts, col])
                    norm = c * 2.0 - 1.0
                    pos = (norm + 1.0) * 0.5 * _SCALE[d]
                    idx = pos.astype(jnp.int32)
                    i0.append(idx)
                    fr.append(pos - idx.astype(jnp.float32))
                base = (
                    i0[0] * _STRIDE[0] + i0[1] * _STRIDE[1]
                    + i0[2] * _STRIDE[2] + i0[3]
                )
                w0 = [1.0 - f for f in fr]
                for cx in range(2):
                    wx = fr[0] if cx else w0[0]
                    for cy in range(2):
                        wxy = wx * (fr[1] if cy else w0[1])
                        for cu in range(2):
                            wxyu = wxy * (fr[2] if cu else w0[2])
                            for cv in range(2):
                                corner = cx | (cy << 1) | (cu << 2) | (cv << 3)
                                off = (cx * _STRIDE[0] + cy * _STRIDE[1]
                                       + cu * _STRIDE[2] + cv)
                                idxbuf[corner, pl.ds(g * L, L)] = base + off
                                w = wxyu * (fr[3] if cv else w0[3])
                                wbuf[corner, pl.ds(g * L, L)] = w
                return c2

            lax.fori_loop(0, G, group_a, 0)

            handles = [
                pltpu.async_copy(
                    table_h.at[idxbuf.at[corner]],
                    rowsbuf.at[pl.ds(corner * P, P)],
                    gsem,
                )
                for corner in range(16)
            ]
            for h in handles:
                h.wait()

            def group_b(g, c2):
                pts = g * L + lane
                acc = [jnp.zeros((L,), jnp.float32) for _ in range(R)]
                for corner in range(16):
                    w = wbuf[corner, pl.ds(g * L, L)]
                    row = corner * P + g * L + lane
                    for r in range(R):
                        col = jnp.full((L,), r, jnp.int32)
                        val = plsc.load_gather(rowsbuf, [row, col])
                        acc[r] = acc[r] + w * val
                for r in range(R):
                    col = jnp.full((L,), r, jnp.int32)
                    plsc.store_scatter(outbuf, [pts, col], acc[r])
                return c2

            lax.fori_loop(0, G, group_b, 0)
            pltpu.sync_copy(outbuf, out_h.at[pl.ds(p0, P)])
            return carry

        lax.fori_loop(0, nchunks, chunk, 0)

    return k(table, xyuv)


_YB = 1


def _tc_transpose(grid6):
    def body(in_ref, out_ref):
        x = in_ref[0, :, 0]
        xr = x.reshape(R, _YB * U * V)
        out_ref[...] = jnp.transpose(xr, (1, 0))

    return pl.pallas_call(
        body,
        grid=(X, Y // _YB),
        in_specs=[
            pl.BlockSpec(
                (1, R, 1, _YB, U, V), lambda i, j: (0, 0, i, j, 0, 0)
            )
        ],
        out_specs=pl.BlockSpec((_YB * U * V, R), lambda i, j: (i * (Y // _YB) + j, 0)),
        out_shape=jax.ShapeDtypeStruct((X * Y * U * V, R), jnp.float32),
    )(grid6)


def kernel(xyuv, xyuv_grid):
    table = _tc_transpose(xyuv_grid)
    return _sc_interp(table, xyuv)


# speedup vs baseline: 1.4751x; 1.1683x over previous
import functools

import jax
import jax.numpy as jnp
from jax import lax
from jax.experimental import pallas as pl
from jax.experimental.pallas import tpu as pltpu
from jax.experimental.pallas import tpu_sc as plsc

R, X, Y, U, V = 8, 16, 16, 128, 128
N = 1048576
L = 16
P = 128
G = P // L

_SCALE = (float(X - 1), float(Y - 1), float(U - 1), float(V - 1))
_STRIDE = (Y * U * V, U * V, V, 1)


def _sc_interp(table, xyuv):
    info = plsc.get_sparse_core_info()
    nw = info.num_cores * info.num_subcores
    pts_per_worker = N // nw
    nchunks = pts_per_worker // P
    npairs = nchunks // 2
    mesh = plsc.VectorSubcoreMesh(core_axis_name="c", subcore_axis_name="s")

    @functools.partial(
        pl.kernel,
        out_type=jax.ShapeDtypeStruct((N, R), jnp.float32),
        mesh=mesh,
        scratch_types=[
            pltpu.VMEM((2, P, 4), jnp.float32),
            pltpu.VMEM((2, 16, P), jnp.int32),
            pltpu.VMEM((2, 16, P), jnp.float32),
            pltpu.VMEM((2, 16 * P, R), jnp.float32),
            pltpu.VMEM((2, P, R), jnp.float32),
            pltpu.SemaphoreType.DMA,
            pltpu.SemaphoreType.DMA,
            pltpu.SemaphoreType.DMA,
            pltpu.SemaphoreType.DMA,
        ],
        compiler_params=pltpu.CompilerParams(
            needs_layout_passes=False, use_tc_tiling_on_sc=False
        ),
    )
    def k(table_h, xyuv_h, out_h, coords, idxbuf, wbuf, rowsbuf, outbuf,
          gsem0, gsem1, osem0, osem1):
        wid = lax.axis_index("s") * info.num_cores + lax.axis_index("c")
        lane = lax.broadcasted_iota(jnp.int32, (L,), 0)
        gsems = (gsem0, gsem1)
        osems = (osem0, osem1)

        def stage_a(c, b):
            p0 = wid * pts_per_worker + c * P
            pltpu.sync_copy(xyuv_h.at[pl.ds(p0, P)], coords.at[b])

            def group_a(g, carry):
                pts = g * L + lane
                fr = []
                i0 = []
                for d in range(4):
                    col = jnp.full((L,), d, jnp.int32)
                    cd = plsc.load_gather(coords.at[b], [pts, col])
                    norm = cd * 2.0 - 1.0
                    pos = (norm + 1.0) * 0.5 * _SCALE[d]
                    idx = pos.astype(jnp.int32)
                    i0.append(idx)
                    fr.append(pos - idx.astype(jnp.float32))
                base = (
                    i0[0] * _STRIDE[0] + i0[1] * _STRIDE[1]
                    + i0[2] * _STRIDE[2] + i0[3]
                )
                w0 = [1.0 - f for f in fr]
                for cx in range(2):
                    wx = fr[0] if cx else w0[0]
                    for cy in range(2):
                        wxy = wx * (fr[1] if cy else w0[1])
                        for cu in range(2):
                            wxyu = wxy * (fr[2] if cu else w0[2])
                            for cv in range(2):
                                corner = cx | (cy << 1) | (cu << 2) | (cv << 3)
                                off = (cx * _STRIDE[0] + cy * _STRIDE[1]
                                       + cu * _STRIDE[2] + cv)
                                idxbuf[b, corner, pl.ds(g * L, L)] = base + off
                                w = wxyu * (fr[3] if cv else w0[3])
                                wbuf[b, corner, pl.ds(g * L, L)] = w
                return carry

            lax.fori_loop(0, G, group_a, 0)

            for corner in range(16):
                pltpu.async_copy(
                    table_h.at[idxbuf.at[b, corner]],
                    rowsbuf.at[b, pl.ds(corner * P, P)],
                    gsems[b],
                )

        def drain_gathers(b):
            for corner in range(16):
                pltpu.make_async_copy(
                    table_h.at[idxbuf.at[b, corner]],
                    rowsbuf.at[b, pl.ds(corner * P, P)],
                    gsems[b],
                ).wait()

        def drain_out(b, p0):
            pltpu.make_async_copy(
                outbuf.at[b], out_h.at[pl.ds(p0, P)], osems[b]
            ).wait()

        def stage_b(c, b, first):
            p0 = wid * pts_per_worker + c * P
            drain_gathers(b)
            if first is None:
                drain_out(b, p0)
            else:
                pl.when(jnp.logical_not(first))(lambda: drain_out(b, p0))

            def group_b(g, carry):
                pts = g * L + lane
                acc = [jnp.zeros((L,), jnp.float32) for _ in range(R)]
                for corner in range(16):
                    w = wbuf[b, corner, pl.ds(g * L, L)]
                    row = corner * P + g * L + lane
                    for r in range(R):
                        col = jnp.full((L,), r, jnp.int32)
                        val = plsc.load_gather(rowsbuf.at[b], [row, col])
                        acc[r] = acc[r] + w * val
                for r in range(R):
                    col = jnp.full((L,), r, jnp.int32)
                    plsc.store_scatter(outbuf.at[b], [pts, col], acc[r])
                return carry

            lax.fori_loop(0, G, group_b, 0)
            pltpu.async_copy(outbuf.at[b], out_h.at[pl.ds(p0, P)], osems[b])

        stage_a(0, 0)

        def pair(g, carry):
            first = g == 0
            c = 2 * g
            stage_a(c + 1, 1)
            stage_b(c, 0, first)
            stage_a(jnp.minimum(c + 2, nchunks - 1), 0)
            stage_b(c + 1, 1, first)
            return carry

        lax.fori_loop(0, npairs, pair, 0)

        drain_gathers(0)
        drain_out(0, wid * pts_per_worker)
        drain_out(1, wid * pts_per_worker)

    return k(table, xyuv)


_YB = 1


def _tc_transpose(grid6):
    def body(in_ref, out_ref):
        x = in_ref[0, :, 0]
        xr = x.reshape(R, _YB * U * V)
        out_ref[...] = jnp.transpose(xr, (1, 0))

    return pl.pallas_call(
        body,
        grid=(X, Y // _YB),
        in_specs=[
            pl.BlockSpec(
                (1, R, 1, _YB, U, V), lambda i, j: (0, 0, i, j, 0, 0)
            )
        ],
        out_specs=pl.BlockSpec((_YB * U * V, R), lambda i, j: (i * (Y // _YB) + j, 0)),
        out_shape=jax.ShapeDtypeStruct((X * Y * U * V, R), jnp.float32),
    )(grid6)


def kernel(xyuv, xyuv_grid):
    table = _tc_transpose(xyuv_grid)
    return _sc_interp(table, xyuv)


# speedup vs baseline: 2.0995x; 1.4233x over previous
import functools

import jax
import jax.numpy as jnp
from jax import lax
from jax.experimental import pallas as pl
from jax.experimental.pallas import tpu as pltpu
from jax.experimental.pallas import tpu_sc as plsc

R, X, Y, U, V = 8, 16, 16, 128, 128
N = 1048576
L = 16
P = 128
G = P // L

_SCALE = (float(X - 1), float(Y - 1), float(U - 1), float(V - 1))
_STRIDE = (Y * U * V, U * V, V, 1)


def _sc_interp(table, xyuv):
    info = plsc.get_sparse_core_info()
    nw = info.num_cores * info.num_subcores
    pts_per_worker = N // nw
    nchunks = pts_per_worker // P
    npairs = nchunks // 2
    mesh = plsc.VectorSubcoreMesh(core_axis_name="c", subcore_axis_name="s")

    @functools.partial(
        pl.kernel,
        out_type=jax.ShapeDtypeStruct((N // P, R, P), jnp.float32),
        mesh=mesh,
        scratch_types=[
            pltpu.VMEM((2, 4, P), jnp.float32),
            pltpu.VMEM((2, 16, P), jnp.int32),
            pltpu.VMEM((2, 16, P), jnp.float32),
            pltpu.VMEM((2, 16 * P, R), jnp.float32),
            pltpu.VMEM((2, R, P), jnp.float32),
            pltpu.SemaphoreType.DMA,
            pltpu.SemaphoreType.DMA,
            pltpu.SemaphoreType.DMA,
            pltpu.SemaphoreType.DMA,
        ],
        compiler_params=pltpu.CompilerParams(
            needs_layout_passes=False, use_tc_tiling_on_sc=False
        ),
    )
    def k(table_h, xyuv_h, out_h, coords, idxbuf, wbuf, rowsbuf, outbuf,
          gsem0, gsem1, osem0, osem1):
        wid = lax.axis_index("s") * info.num_cores + lax.axis_index("c")
        lane = lax.broadcasted_iota(jnp.int32, (L,), 0)
        gsems = (gsem0, gsem1)
        osems = (osem0, osem1)

        def stage_a(c, b):
            blk = wid * nchunks + c
            pltpu.sync_copy(xyuv_h.at[blk], coords.at[b])

            def group_a(g, carry):
                fr = []
                i0 = []
                for d in range(4):
                    cd = coords[b, d, pl.ds(g * L, L)]
                    norm = cd * 2.0 - 1.0
                    pos = (norm + 1.0) * 0.5 * _SCALE[d]
                    idx = pos.astype(jnp.int32)
                    i0.append(idx)
                    fr.append(pos - idx.astype(jnp.float32))
                base = (
                    i0[0] * _STRIDE[0] + i0[1] * _STRIDE[1]
                    + i0[2] * _STRIDE[2] + i0[3]
                )
                w0 = [1.0 - f for f in fr]
                for cx in range(2):
                    wx = fr[0] if cx else w0[0]
                    for cy in range(2):
                        wxy = wx * (fr[1] if cy else w0[1])
                        for cu in range(2):
                            wxyu = wxy * (fr[2] if cu else w0[2])
                            for cv in range(2):
                                corner = cx | (cy << 1) | (cu << 2) | (cv << 3)
                                off = (cx * _STRIDE[0] + cy * _STRIDE[1]
                                       + cu * _STRIDE[2] + cv)
                                idxbuf[b, corner, pl.ds(g * L, L)] = base + off
                                w = wxyu * (fr[3] if cv else w0[3])
                                wbuf[b, corner, pl.ds(g * L, L)] = w
                return carry

            lax.fori_loop(0, G, group_a, 0)

            for corner in range(16):
                pltpu.async_copy(
                    table_h.at[idxbuf.at[b, corner]],
                    rowsbuf.at[b, pl.ds(corner * P, P)],
                    gsems[b],
                )

        def drain_gathers(b):
            for corner in range(16):
                pltpu.make_async_copy(
                    table_h.at[idxbuf.at[b, corner]],
                    rowsbuf.at[b, pl.ds(corner * P, P)],
                    gsems[b],
                ).wait()

        def drain_out(b, blk):
            pltpu.make_async_copy(
                outbuf.at[b], out_h.at[blk], osems[b]
            ).wait()

        def stage_b(c, b, first):
            blk = wid * nchunks + c
            drain_gathers(b)
            pl.when(jnp.logical_not(first))(lambda: drain_out(b, blk))

            def group_b(g, carry):
                acc = [jnp.zeros((L,), jnp.float32) for _ in range(R)]
                for corner in range(16):
                    w = wbuf[b, corner, pl.ds(g * L, L)]
                    row = corner * P + g * L + lane
                    for r in range(R):
                        col = jnp.full((L,), r, jnp.int32)
                        val = plsc.load_gather(rowsbuf.at[b], [row, col])
                        acc[r] = acc[r] + w * val
                for r in range(R):
                    outbuf[b, r, pl.ds(g * L, L)] = acc[r]
                return carry

            lax.fori_loop(0, G, group_b, 0)
            pltpu.async_copy(outbuf.at[b], out_h.at[blk], osems[b])

        stage_a(0, 0)

        def pair(g, carry):
            first = g == 0
            c = 2 * g
            stage_a(c + 1, 1)
            stage_b(c, 0, first)
            stage_a(jnp.minimum(c + 2, nchunks - 1), 0)
            stage_b(c + 1, 1, first)
            return carry

        lax.fori_loop(0, npairs, pair, 0)

        drain_gathers(0)
        drain_out(0, wid * nchunks)
        drain_out(1, wid * nchunks)

    return k(table, xyuv)


_M = X * Y * U * V
_LPB = 128
_CPB = 16 * _LPB
_TU = 2048
_TL = _TU // 2


def _tc_build_table(grid2):
    def k(in_ref, out_ref):
        x = in_ref[...]
        y = x.reshape(R, _LPB, 16).transpose(1, 2, 0).reshape(_LPB, 128)
        out_ref[...] = y

    return pl.pallas_call(
        k,
        grid=(_M // _CPB,),
        in_specs=[pl.BlockSpec((R, _CPB), lambda i: (0, i))],
        out_specs=pl.BlockSpec((_LPB, 128), lambda i: (i, 0)),
        out_shape=jax.ShapeDtypeStruct((_M // 16, 128), jnp.float32),
    )(grid2)


def _sc_build_table(grid2):
    info = plsc.get_sparse_core_info()
    nw = info.num_cores * info.num_subcores
    upw = (_M // _TU) // nw
    mesh = plsc.VectorSubcoreMesh(core_axis_name="c", subcore_axis_name="s")

    @functools.partial(
        pl.kernel,
        out_type=jax.ShapeDtypeStruct((_M // 2, 16), jnp.float32),
        mesh=mesh,
        scratch_types=[
            pltpu.VMEM((2, R, _TU), jnp.float32),
            pltpu.VMEM((2, _TL, 16), jnp.float32),
            pltpu.SemaphoreType.DMA,
            pltpu.SemaphoreType.DMA,
            pltpu.SemaphoreType.DMA,
            pltpu.SemaphoreType.DMA,
        ],
        compiler_params=pltpu.CompilerParams(
            needs_layout_passes=False, use_tc_tiling_on_sc=False
        ),
    )
    def k(grid_h, out_h, inbuf, outb, i0, i1, o0, o1):
        wid = lax.axis_index("s") * info.num_cores + lax.axis_index("c")
        isems = (i0, i1)
        osems = (o0, o1)
        lane = lax.broadcasted_iota(jnp.int32, (L,), 0)
        ridx = lane % 8
        voff = lane // 8

        def fire_in(u, b):
            base = (wid * upw + u) * _TU
            for r in range(R):
                pltpu.async_copy(
                    grid_h.at[r, pl.ds(base, _TU)], inbuf.at[b, r], isems[b]
                )

        def drain_in(b):
            for r in range(R):
                pltpu.make_async_copy(
                    grid_h.at[r, pl.ds(0, _TU)], inbuf.at[b, r], isems[b]
                ).wait()

        def drain_outc(b):
            pltpu.make_async_copy(
                outb.at[b], out_h.at[pl.ds(0, _TL)], osems[b]
            ).wait()

        def unit(u, b, first):
            drain_in(b)
            pl.when(jnp.logical_not(first))(lambda: drain_outc(b))

            def lines(j8, carry):
                for jj in range(8):
                    j = j8 * 8 + jj
                    val = plsc.load_gather(inbuf.at[b], [ridx, 2 * j + voff])
                    outb[b, j, :] = val
                return carry

            lax.fori_loop(0, _TL // 8, lines, 0)
            base_l = (wid * upw + u) * _TL
            pltpu.async_copy(
                outb.at[b], out_h.at[pl.ds(base_l, _TL)], osems[b]
            )

        fire_in(0, 0)

        def pair(g, carry):
            first = g == 0
            u0 = 2 * g
            fire_in(u0 + 1, 1)
            unit(u0, 0, first)
            fire_in(jnp.minimum(u0 + 2, upw - 1), 0)
            unit(u0 + 1, 1, first)
            return carry

        lax.fori_loop(0, upw // 2, pair, 0)

        drain_in(0)
        drain_outc(0)
        drain_outc(1)

    return k(grid2)


def kernel(xyuv, xyuv_grid):
    grid2 = xyuv_grid.reshape(R, _M)
    table = _tc_build_table(grid2).reshape(_M, R)
    xyuv_b = xyuv.reshape(N // P, P, 4).swapaxes(1, 2)
    out_b = _sc_interp(table, xyuv_b)
    return out_b.swapaxes(1, 2).reshape(N, R)


# speedup vs baseline: 5.0383x; 2.3998x over previous
import functools

import jax
import jax.numpy as jnp
from jax import lax
from jax.experimental import pallas as pl
from jax.experimental.pallas import tpu as pltpu
from jax.experimental.pallas import tpu_sc as plsc

R, X, Y, U, V = 8, 16, 16, 128, 128
N = 1048576
L = 16
P = 128
G = P // L

_SCALE = (float(X - 1), float(Y - 1), float(U - 1), float(V - 1))
_STRIDE = (Y * U * V, U * V, V, 1)


def _sc_interp(table, xyuv):
    info = plsc.get_sparse_core_info()
    nw = info.num_cores * info.num_subcores
    pts_per_worker = N // nw
    nchunks = pts_per_worker // P
    npairs = nchunks // 2
    mesh = plsc.VectorSubcoreMesh(core_axis_name="c", subcore_axis_name="s")

    @functools.partial(
        pl.kernel,
        out_type=jax.ShapeDtypeStruct((N // P, R, P), jnp.float32),
        mesh=mesh,
        scratch_types=[
            pltpu.VMEM((2, 4, P), jnp.float32),
            pltpu.VMEM((2, 16, P), jnp.int32),
            pltpu.VMEM((2, 16, P), jnp.float32),
            pltpu.VMEM((2, 16 * P, R), jnp.float32),
            pltpu.VMEM((2, R, P), jnp.float32),
            pltpu.SemaphoreType.DMA,
            pltpu.SemaphoreType.DMA,
            pltpu.SemaphoreType.DMA,
            pltpu.SemaphoreType.DMA,
        ],
        compiler_params=pltpu.CompilerParams(
            needs_layout_passes=False, use_tc_tiling_on_sc=False
        ),
    )
    def k(table_h, xyuv_h, out_h, coords, idxbuf, wbuf, rowsbuf, outbuf,
          gsem0, gsem1, osem0, osem1):
        wid = lax.axis_index("s") * info.num_cores + lax.axis_index("c")
        lane = lax.broadcasted_iota(jnp.int32, (L,), 0)
        gsems = (gsem0, gsem1)
        osems = (osem0, osem1)

        def stage_a(c, b):
            blk = wid * nchunks + c
            pltpu.sync_copy(xyuv_h.at[blk], coords.at[b])

            def group_a(g, carry):
                fr = []
                i0 = []
                for d in range(4):
                    cd = coords[b, d, pl.ds(g * L, L)]
                    norm = cd * 2.0 - 1.0
                    pos = (norm + 1.0) * 0.5 * _SCALE[d]
                    idx = pos.astype(jnp.int32)
                    i0.append(idx)
                    fr.append(pos - idx.astype(jnp.float32))
                base = (
                    i0[0] * _STRIDE[0] + i0[1] * _STRIDE[1]
                    + i0[2] * _STRIDE[2] + i0[3]
                )
                w0 = [1.0 - f for f in fr]
                for cx in range(2):
                    wx = fr[0] if cx else w0[0]
                    for cy in range(2):
                        wxy = wx * (fr[1] if cy else w0[1])
                        for cu in range(2):
                            wxyu = wxy * (fr[2] if cu else w0[2])
                            for cv in range(2):
                                corner = cx | (cy << 1) | (cu << 2) | (cv << 3)
                                off = (cx * _STRIDE[0] + cy * _STRIDE[1]
                                       + cu * _STRIDE[2] + cv)
                                idxbuf[b, corner, pl.ds(g * L, L)] = base + off
                                w = wxyu * (fr[3] if cv else w0[3])
                                wbuf[b, corner, pl.ds(g * L, L)] = w
                return carry

            lax.fori_loop(0, G, group_a, 0)

            for corner in range(16):
                pltpu.async_copy(
                    table_h.at[idxbuf.at[b, corner]],
                    rowsbuf.at[b, pl.ds(corner * P, P)],
                    gsems[b],
                )

        def drain_gathers(b):
            for corner in range(16):
                pltpu.make_async_copy(
                    table_h.at[idxbuf.at[b, corner]],
                    rowsbuf.at[b, pl.ds(corner * P, P)],
                    gsems[b],
                ).wait()

        def drain_out(b, blk):
            pltpu.make_async_copy(
                outbuf.at[b], out_h.at[blk], osems[b]
            ).wait()

        def stage_b(c, b, first):
            blk = wid * nchunks + c
            drain_gathers(b)
            pl.when(jnp.logical_not(first))(lambda: drain_out(b, blk))

            def group_b(g, carry):
                acc = [jnp.zeros((L,), jnp.float32) for _ in range(R)]
                for corner in range(16):
                    w = wbuf[b, corner, pl.ds(g * L, L)]
                    row = corner * P + g * L + lane
                    for r in range(R):
                        col = jnp.full((L,), r, jnp.int32)
                        val = plsc.load_gather(rowsbuf.at[b], [row, col])
                        acc[r] = acc[r] + w * val
                for r in range(R):
                    outbuf[b, r, pl.ds(g * L, L)] = acc[r]
                return carry

            lax.fori_loop(0, G, group_b, 0)
            pltpu.async_copy(outbuf.at[b], out_h.at[blk], osems[b])

        stage_a(0, 0)

        def pair(g, carry):
            first = g == 0
            c = 2 * g
            stage_a(c + 1, 1)
            stage_b(c, 0, first)
            stage_a(jnp.minimum(c + 2, nchunks - 1), 0)
            stage_b(c + 1, 1, first)
            return carry

        lax.fori_loop(0, npairs, pair, 0)

        drain_gathers(0)
        drain_out(0, wid * nchunks)
        drain_out(1, wid * nchunks)

    return k(table, xyuv)


_M = X * Y * U * V
_TU = 2048
_TL = _TU // 2


def _sc_build_table(grid2):
    info = plsc.get_sparse_core_info()
    nw = info.num_cores * info.num_subcores
    upw = (_M // _TU) // nw
    mesh = plsc.VectorSubcoreMesh(core_axis_name="c", subcore_axis_name="s")

    @functools.partial(
        pl.kernel,
        out_type=jax.ShapeDtypeStruct((_M // 2, 16), jnp.float32),
        mesh=mesh,
        scratch_types=[
            pltpu.VMEM((2, R, _TU), jnp.float32),
            pltpu.VMEM((2, _TL, 16), jnp.float32),
            pltpu.SemaphoreType.DMA,
            pltpu.SemaphoreType.DMA,
            pltpu.SemaphoreType.DMA,
            pltpu.SemaphoreType.DMA,
        ],
        compiler_params=pltpu.CompilerParams(
            needs_layout_passes=False, use_tc_tiling_on_sc=False
        ),
    )
    def k(grid_h, out_h, inbuf, outb, i0, i1, o0, o1):
        wid = lax.axis_index("s") * info.num_cores + lax.axis_index("c")
        isems = (i0, i1)
        osems = (o0, o1)
        lane = lax.broadcasted_iota(jnp.int32, (L,), 0)
        ridx = lane % 8
        voff = lane // 8

        def fire_in(u, b):
            base = (wid * upw + u) * _TU
            for r in range(R):
                pltpu.async_copy(
                    grid_h.at[r, pl.ds(base, _TU)], inbuf.at[b, r], isems[b]
                )

        def drain_in(b):
            for r in range(R):
                pltpu.make_async_copy(
                    grid_h.at[r, pl.ds(0, _TU)], inbuf.at[b, r], isems[b]
                ).wait()

        def drain_outc(b):
            pltpu.make_async_copy(
                outb.at[b], out_h.at[pl.ds(0, _TL)], osems[b]
            ).wait()

        def unit(u, b, first):
            drain_in(b)
            pl.when(jnp.logical_not(first))(lambda: drain_outc(b))

            def lines(j8, carry):
                for jj in range(8):
                    j = j8 * 8 + jj
                    val = plsc.load_gather(inbuf.at[b], [ridx, 2 * j + voff])
                    outb[b, j, :] = val
                return carry

            lax.fori_loop(0, _TL // 8, lines, 0)
            base_l = (wid * upw + u) * _TL
            pltpu.async_copy(
                outb.at[b], out_h.at[pl.ds(base_l, _TL)], osems[b]
            )

        fire_in(0, 0)

        def pair(g, carry):
            first = g == 0
            u0 = 2 * g
            fire_in(u0 + 1, 1)
            unit(u0, 0, first)
            fire_in(jnp.minimum(u0 + 2, upw - 1), 0)
            unit(u0 + 1, 1, first)
            return carry

        lax.fori_loop(0, upw // 2, pair, 0)

        drain_in(0)
        drain_outc(0)
        drain_outc(1)

    return k(grid2)


def kernel(xyuv, xyuv_grid):
    grid2 = xyuv_grid.reshape(R, _M)
    table = _sc_build_table(grid2).reshape(_M, R)
    xyuv_b = xyuv.reshape(N // P, P, 4).swapaxes(1, 2)
    out_b = _sc_interp(table, xyuv_b)
    return out_b.swapaxes(1, 2).reshape(N, R)
